# baseline (speedup 1.0000x reference)
import jax
import jax.numpy as jnp
from jax.experimental import pallas as pl
from jax.experimental.pallas import tpu as pltpu

_NB = 1000
_NSPLIT = 32
_PAD = 4


def _box9(x, axis):
    n = x.shape[axis]
    zshape = list(x.shape)
    zshape[axis] = _PAD
    z = jnp.zeros(zshape, x.dtype)
    xp = jnp.concatenate([z, x, z], axis=axis)

    def sl(a, lo, hi):
        if axis == 0:
            return a[lo:hi]
        return a[:, lo:hi]

    m = n + 8
    w2 = sl(xp, 0, m - 1) + sl(xp, 1, m)
    w4 = sl(w2, 0, m - 3) + sl(w2, 2, m - 1)
    w8 = sl(w4, 0, m - 7) + sl(w4, 4, m - 3)
    return sl(w8, 0, n) + sl(xp, 8, m)


def _dist(x):
    h, w = x.shape
    s = _box9(_box9(x, 0), 1)
    ri = jax.lax.broadcasted_iota(jnp.int32, (h, w), 0).astype(jnp.float32)
    ci = jax.lax.broadcasted_iota(jnp.int32, (h, w), 1).astype(jnp.float32)
    cr = jnp.minimum(ri + 4.0, h - 1.0) - jnp.maximum(ri - 4.0, 0.0) + 1.0
    cc = jnp.minimum(ci + 4.0, w - 1.0) - jnp.maximum(ci - 4.0, 0.0) + 1.0
    return x - s / (cr * cc)


def _pass_a_kernel(p_ref, t_ref, dp_ref, dt_ref, mn_ref, mx_ref):
    dp = _dist(p_ref[0])
    dt = _dist(t_ref[0])
    dp_ref[0] = dp
    dt_ref[0] = dt
    mn = jnp.minimum(jnp.min(dp), jnp.min(dt))
    mx = jnp.maximum(jnp.max(dp), jnp.max(dt))
    mn_ref[...] = jnp.full((1, 1, 128), mn, jnp.float32)
    mx_ref[...] = jnp.full((1, 1, 128), mx, jnp.float32)


def _onehot_slab(x, mn, scale, iota_s):
    t = (x - mn) * scale
    bidx = jnp.clip(jnp.ceil(t), 0.0, _NB - 1.0)
    c = jnp.floor(bidx * (1.0 / _NSPLIT))
    f = bidx - _NSPLIT * c
    cb = c.astype(jnp.bfloat16)
    fb = f.astype(jnp.bfloat16)
    one = jnp.bfloat16(1.0)
    zero = jnp.bfloat16(0.0)
    u_rows = []
    v_rows = []
    for g in range(8):
        cg = cb[g:g + 1, :]
        fg = fb[g:g + 1, :]
        u_rows.append(jnp.where(cg == iota_s, one, zero))
        v_rows.append(jnp.where(fg == iota_s, one, zero))
    u = jnp.concatenate(u_rows, axis=0).astype(jnp.float8_e4m3fn)
    v = jnp.concatenate(v_rows, axis=0)
    return u, v


def _mxu_accumulate(u, v, mxu_index):
    w = u.shape[1]
    for k in range(w // 256):
        vk = v[:, k * 256:(k + 1) * 256].T.astype(jnp.float8_e4m3fn)
        uk = u[:, k * 256:(k + 1) * 256]
        sreg = k % 2
        pltpu.matmul_push_rhs(vk, sreg, mxu_index, transpose=False)
        pltpu.matmul_acc_lhs(0, uk, mxu_index, load_staged_rhs=sreg)


def _diag_hist(acc):
    h = acc[0:_NSPLIT, 0:_NSPLIT]
    for g in range(1, 8):
        lo = g * _NSPLIT
        h = h + acc[lo:lo + _NSPLIT, lo:lo + _NSPLIT]
    return h


def _shift_sum(x, axis):
    n = x.shape[axis]
    k = 1
    while k < n:
        if axis == 0:
            z = jnp.zeros((k,) + x.shape[1:], x.dtype)
            x = x + jnp.concatenate([z, x[:-k]], axis=0)
        else:
            z = jnp.zeros(x.shape[:1] + (k,), x.dtype)
            x = x + jnp.concatenate([z, x[:, :-k]], axis=1)
        k *= 2
    return x


def _cum2d(h):
    a = _shift_sum(h, 1)
    tot = a[:, _NSPLIT - 1:_NSPLIT]
    ctot = _shift_sum(tot, 0)
    return a + (ctot - tot)


def _pass_b_kernel(dp_ref, dt_ref, mn_ref, mx_ref, out_ref):
    ci = pl.program_id(1)
    nchunk = pl.num_programs(1)
    rows = dp_ref.shape[1]
    w = dp_ref.shape[2]
    n_elems = float(rows * nchunk * w)

    bi = pl.program_id(0)

    @pl.when(jnp.logical_and(bi == 0, ci == 0))
    def _():
        z = pltpu.matmul_pop(0, (256, 256), jnp.float32, 0)
        z2 = pltpu.matmul_pop(0, (256, 256), jnp.float32, 1)
        out_ref[...] = jnp.full((1, 1, 128), 0.0 * (z[0, 0] + z2[0, 0]), jnp.float32)

    mn = mn_ref[0, 0, 0]
    mx = mx_ref[0, 0, 0]
    scale = (_NB - 1.0) / jnp.maximum(mx - mn, 1e-30)
    iota_s = jax.lax.broadcasted_iota(jnp.int32, (_NSPLIT, w), 0).astype(jnp.bfloat16)

    def body(s, _):
        up, vp = _onehot_slab(dp_ref[0, pl.ds(s * 8, 8), :], mn, scale, iota_s)
        ut, vt = _onehot_slab(dt_ref[0, pl.ds(s * 8, 8), :], mn, scale, iota_s)
        _mxu_accumulate(up, vp, 0)
        _mxu_accumulate(ut, vt, 1)
        return 0

    jax.lax.fori_loop(0, rows // 8, body, 0)

    @pl.when(ci == nchunk - 1)
    def _():
        hp = _diag_hist(pltpu.matmul_pop(0, (256, 256), jnp.float32, 0))
        ht = _diag_hist(pltpu.matmul_pop(0, (256, 256), jnp.float32, 1))
        cump = _cum2d(hp)
        cumt = _cum2d(ht)
        fi = jax.lax.broadcasted_iota(jnp.int32, (_NSPLIT, _NSPLIT), 0).astype(jnp.float32)
        fj = jax.lax.broadcasted_iota(jnp.int32, (_NSPLIT, _NSPLIT), 1).astype(jnp.float32)
        dd = (cump - cumt) * (1.0 / n_elems)
        d2 = dd * dd
        lin = _NSPLIT * fi + fj
        total = jnp.sum(jnp.where(lin <= _NB - 1.0, d2, 0.0))
        first = jnp.sum(jnp.where(lin == 0.0, d2, 0.0))
        last = jnp.sum(jnp.where(lin == _NB - 1.0, d2, 0.0))
        dx = (mx - mn) * (1.0 / (_NB - 1.0))
        crps = dx * (total - 0.5 * (first + last))
        out_ref[...] = jnp.full((1, 1, 128), crps, jnp.float32)


def kernel(prediction, target):
    b, _, h, w = prediction.shape
    p = prediction.reshape(b, h, w)
    t = target.reshape(b, h, w)

    dp, dt, mn, mx = pl.pallas_call(
        _pass_a_kernel,
        out_shape=(
            jax.ShapeDtypeStruct((b, h, w), jnp.float32),
            jax.ShapeDtypeStruct((b, h, w), jnp.float32),
            jax.ShapeDtypeStruct((b, 1, 128), jnp.float32),
            jax.ShapeDtypeStruct((b, 1, 128), jnp.float32),
        ),
        grid=(b,),
        in_specs=[
            pl.BlockSpec((1, h, w), lambda i: (i, 0, 0)),
            pl.BlockSpec((1, h, w), lambda i: (i, 0, 0)),
        ],
        out_specs=(
            pl.BlockSpec((1, h, w), lambda i: (i, 0, 0)),
            pl.BlockSpec((1, h, w), lambda i: (i, 0, 0)),
            pl.BlockSpec((1, 1, 128), lambda i: (i, 0, 0)),
            pl.BlockSpec((1, 1, 128), lambda i: (i, 0, 0)),
        ),
        compiler_params=pltpu.CompilerParams(
            dimension_semantics=("parallel",),
            vmem_limit_bytes=56 * 1024 * 1024,
        ),
        name="crps_pool_dist",
    )(p, t)

    rows = 128
    nchunk = h // rows
    crps = pl.pallas_call(
        _pass_b_kernel,
        out_shape=jax.ShapeDtypeStruct((b, 1, 128), jnp.float32),
        grid=(b, nchunk),
        in_specs=[
            pl.BlockSpec((1, rows, w), lambda i, j: (i, j, 0)),
            pl.BlockSpec((1, rows, w), lambda i, j: (i, j, 0)),
            pl.BlockSpec((1, 1, 128), lambda i, j: (i, 0, 0)),
            pl.BlockSpec((1, 1, 128), lambda i, j: (i, 0, 0)),
        ],
        out_specs=pl.BlockSpec((1, 1, 128), lambda i, j: (i, 0, 0)),
        compiler_params=pltpu.CompilerParams(
            dimension_semantics=("parallel", "arbitrary"),
            vmem_limit_bytes=56 * 1024 * 1024,
        ),
        name="crps_hist",
    )(dp, dt, mn, mx)

    return jnp.mean(crps[:, 0, 0])


# speedup vs baseline: 1.3088x; 1.3088x over previous
import jax
import jax.numpy as jnp
from jax.experimental import pallas as pl
from jax.experimental.pallas import tpu as pltpu

_NB = 1000
_NSPLIT = 32
_PAD = 4


def _box9(x, axis):
    n = x.shape[axis]
    zshape = list(x.shape)
    zshape[axis] = _PAD
    z = jnp.zeros(zshape, x.dtype)
    xp = jnp.concatenate([z, x, z], axis=axis)

    def sl(a, lo, hi):
        if axis == 0:
            return a[lo:hi]
        return a[:, lo:hi]

    m = n + 8
    w2 = sl(xp, 0, m - 1) + sl(xp, 1, m)
    w4 = sl(w2, 0, m - 3) + sl(w2, 2, m - 1)
    w8 = sl(w4, 0, m - 7) + sl(w4, 4, m - 3)
    return sl(w8, 0, n) + sl(xp, 8, m)


def _dist(x):
    h, w = x.shape
    s = _box9(_box9(x, 0), 1)
    ri = jax.lax.broadcasted_iota(jnp.int32, (h, w), 0).astype(jnp.float32)
    ci = jax.lax.broadcasted_iota(jnp.int32, (h, w), 1).astype(jnp.float32)
    cr = jnp.minimum(ri + 4.0, h - 1.0) - jnp.maximum(ri - 4.0, 0.0) + 1.0
    cc = jnp.minimum(ci + 4.0, w - 1.0) - jnp.maximum(ci - 4.0, 0.0) + 1.0
    return x - s / (cr * cc)


def _pass_a_kernel(p_ref, t_ref, dp_ref, dt_ref, mn_ref, mx_ref):
    dp = _dist(p_ref[0])
    dt = _dist(t_ref[0])
    dp_ref[0] = dp
    dt_ref[0] = dt
    mn = jnp.minimum(jnp.min(dp), jnp.min(dt))
    mx = jnp.maximum(jnp.max(dp), jnp.max(dt))
    mn_ref[...] = jnp.full((1, 1, 128), mn, jnp.float32)
    mx_ref[...] = jnp.full((1, 1, 128), mx, jnp.float32)


def _onehot_slab(x, mn, scale, iota_s):
    t = (x - mn) * scale
    bidx = jnp.clip(jnp.ceil(t), 0.0, _NB - 1.0)
    c = jnp.floor(bidx * (1.0 / _NSPLIT))
    f = bidx - _NSPLIT * c
    cb = c.astype(jnp.bfloat16)
    fb = f.astype(jnp.bfloat16)
    one = jnp.bfloat16(1.0)
    zero = jnp.bfloat16(0.0)
    u_rows = []
    v_rows = []
    for g in range(8):
        cg = cb[g:g + 1, :]
        fg = fb[g:g + 1, :]
        u_rows.append(jnp.where(cg == iota_s, one, zero))
        v_rows.append(jnp.where(fg == iota_s, one, zero))
    u = jnp.concatenate(u_rows, axis=0).astype(jnp.float8_e4m3fn)
    v = jnp.concatenate(v_rows, axis=0).astype(jnp.float8_e4m3fn)
    return u, v


def _mxu_accumulate(u, v, mxu_index):
    w = u.shape[1]
    for k in range(w // 256):
        vk = v[:, k * 256:(k + 1) * 256]
        uk = u[:, k * 256:(k + 1) * 256]
        sreg = k % 2
        pltpu.matmul_push_rhs(vk, sreg, mxu_index, transpose=True)
        pltpu.matmul_acc_lhs(0, uk, mxu_index, load_staged_rhs=sreg)


def _diag_hist(acc):
    h = acc[0:_NSPLIT, 0:_NSPLIT]
    for g in range(1, 8):
        lo = g * _NSPLIT
        h = h + acc[lo:lo + _NSPLIT, lo:lo + _NSPLIT]
    return h


def _shift_sum(x, axis):
    n = x.shape[axis]
    k = 1
    while k < n:
        if axis == 0:
            z = jnp.zeros((k,) + x.shape[1:], x.dtype)
            x = x + jnp.concatenate([z, x[:-k]], axis=0)
        else:
            z = jnp.zeros(x.shape[:1] + (k,), x.dtype)
            x = x + jnp.concatenate([z, x[:, :-k]], axis=1)
        k *= 2
    return x


def _cum2d(h):
    a = _shift_sum(h, 1)
    tot = a[:, _NSPLIT - 1:_NSPLIT]
    ctot = _shift_sum(tot, 0)
    return a + (ctot - tot)


def _pass_b_kernel(dp_ref, dt_ref, mn_ref, mx_ref, out_ref):
    ci = pl.program_id(1)
    nchunk = pl.num_programs(1)
    rows = dp_ref.shape[1]
    w = dp_ref.shape[2]
    n_elems = float(rows * nchunk * w)

    bi = pl.program_id(0)

    @pl.when(jnp.logical_and(bi == 0, ci == 0))
    def _():
        z = pltpu.matmul_pop(0, (256, 256), jnp.float32, 0)
        z2 = pltpu.matmul_pop(0, (256, 256), jnp.float32, 1)
        out_ref[...] = jnp.full((1, 1, 128), 0.0 * (z[0, 0] + z2[0, 0]), jnp.float32)

    mn = mn_ref[0, 0, 0]
    mx = mx_ref[0, 0, 0]
    scale = (_NB - 1.0) / jnp.maximum(mx - mn, 1e-30)
    iota_s = jax.lax.broadcasted_iota(jnp.int32, (_NSPLIT, w), 0).astype(jnp.bfloat16)

    def body(s, _):
        up, vp = _onehot_slab(dp_ref[0, pl.ds(s * 8, 8), :], mn, scale, iota_s)
        ut, vt = _onehot_slab(dt_ref[0, pl.ds(s * 8, 8), :], mn, scale, iota_s)
        _mxu_accumulate(up, vp, 0)
        _mxu_accumulate(ut, vt, 1)
        return 0

    jax.lax.fori_loop(0, rows // 8, body, 0)

    @pl.when(ci == nchunk - 1)
    def _():
        hp = _diag_hist(pltpu.matmul_pop(0, (256, 256), jnp.float32, 0))
        ht = _diag_hist(pltpu.matmul_pop(0, (256, 256), jnp.float32, 1))
        cump = _cum2d(hp)
        cumt = _cum2d(ht)
        fi = jax.lax.broadcasted_iota(jnp.int32, (_NSPLIT, _NSPLIT), 0).astype(jnp.float32)
        fj = jax.lax.broadcasted_iota(jnp.int32, (_NSPLIT, _NSPLIT), 1).astype(jnp.float32)
        dd = (cump - cumt) * (1.0 / n_elems)
        d2 = dd * dd
        lin = _NSPLIT * fi + fj
        total = jnp.sum(jnp.where(lin <= _NB - 1.0, d2, 0.0))
        first = jnp.sum(jnp.where(lin == 0.0, d2, 0.0))
        last = jnp.sum(jnp.where(lin == _NB - 1.0, d2, 0.0))
        dx = (mx - mn) * (1.0 / (_NB - 1.0))
        crps = dx * (total - 0.5 * (first + last))
        out_ref[...] = jnp.full((1, 1, 128), crps, jnp.float32)


def kernel(prediction, target):
    b, _, h, w = prediction.shape
    p = prediction.reshape(b, h, w)
    t = target.reshape(b, h, w)

    dp, dt, mn, mx = pl.pallas_call(
        _pass_a_kernel,
        out_shape=(
            jax.ShapeDtypeStruct((b, h, w), jnp.float32),
            jax.ShapeDtypeStruct((b, h, w), jnp.float32),
            jax.ShapeDtypeStruct((b, 1, 128), jnp.float32),
            jax.ShapeDtypeStruct((b, 1, 128), jnp.float32),
        ),
        grid=(b,),
        in_specs=[
            pl.BlockSpec((1, h, w), lambda i: (i, 0, 0)),
            pl.BlockSpec((1, h, w), lambda i: (i, 0, 0)),
        ],
        out_specs=(
            pl.BlockSpec((1, h, w), lambda i: (i, 0, 0)),
            pl.BlockSpec((1, h, w), lambda i: (i, 0, 0)),
            pl.BlockSpec((1, 1, 128), lambda i: (i, 0, 0)),
            pl.BlockSpec((1, 1, 128), lambda i: (i, 0, 0)),
        ),
        compiler_params=pltpu.CompilerParams(
            dimension_semantics=("parallel",),
            vmem_limit_bytes=56 * 1024 * 1024,
        ),
        name="crps_pool_dist",
    )(p, t)

    rows = 128
    nchunk = h // rows
    crps = pl.pallas_call(
        _pass_b_kernel,
        out_shape=jax.ShapeDtypeStruct((b, 1, 128), jnp.float32),
        grid=(b, nchunk),
        in_specs=[
            pl.BlockSpec((1, rows, w), lambda i, j: (i, j, 0)),
            pl.BlockSpec((1, rows, w), lambda i, j: (i, j, 0)),
            pl.BlockSpec((1, 1, 128), lambda i, j: (i, 0, 0)),
            pl.BlockSpec((1, 1, 128), lambda i, j: (i, 0, 0)),
        ],
        out_specs=pl.BlockSpec((1, 1, 128), lambda i, j: (i, 0, 0)),
        compiler_params=pltpu.CompilerParams(
            dimension_semantics=("parallel", "arbitrary"),
            vmem_limit_bytes=56 * 1024 * 1024,
        ),
        name="crps_hist",
    )(dp, dt, mn, mx)

    return jnp.mean(crps[:, 0, 0])


# speedup vs baseline: 1.3265x; 1.0135x over previous
import jax
import jax.numpy as jnp
from jax.experimental import pallas as pl
from jax.experimental.pallas import tpu as pltpu

_NB = 1000
_NSPLIT = 32
_PAD = 4


def _box9(x, axis):
    n = x.shape[axis]
    zshape = list(x.shape)
    zshape[axis] = _PAD
    z = jnp.zeros(zshape, x.dtype)
    xp = jnp.concatenate([z, x, z], axis=axis)

    def sl(a, lo, hi):
        if axis == 0:
            return a[lo:hi]
        return a[:, lo:hi]

    m = n + 8
    w2 = sl(xp, 0, m - 1) + sl(xp, 1, m)
    w4 = sl(w2, 0, m - 3) + sl(w2, 2, m - 1)
    w8 = sl(w4, 0, m - 7) + sl(w4, 4, m - 3)
    return sl(w8, 0, n) + sl(xp, 8, m)


def _dist(x):
    h, w = x.shape
    s = _box9(_box9(x, 0), 1)
    ri = jax.lax.broadcasted_iota(jnp.int32, (h, w), 0).astype(jnp.float32)
    ci = jax.lax.broadcasted_iota(jnp.int32, (h, w), 1).astype(jnp.float32)
    cr = jnp.minimum(ri + 4.0, h - 1.0) - jnp.maximum(ri - 4.0, 0.0) + 1.0
    cc = jnp.minimum(ci + 4.0, w - 1.0) - jnp.maximum(ci - 4.0, 0.0) + 1.0
    return x - s / (cr * cc)


def _pass_a_kernel(p_ref, t_ref, dp_ref, dt_ref, mn_ref, mx_ref):
    dp = _dist(p_ref[0])
    dt = _dist(t_ref[0])
    dp_ref[0] = dp
    dt_ref[0] = dt
    mn = jnp.minimum(jnp.min(dp), jnp.min(dt))
    mx = jnp.maximum(jnp.max(dp), jnp.max(dt))
    mn_ref[...] = jnp.full((1, 1, 128), mn, jnp.float32)
    mx_ref[...] = jnp.full((1, 1, 128), mx, jnp.float32)


def _onehot_slab(x, mn, scale, iota_s):
    t = (x - mn) * scale
    bidx = jnp.clip(jnp.ceil(t), 0.0, _NB - 1.0)
    c = jnp.floor(bidx * (1.0 / _NSPLIT))
    f = bidx - _NSPLIT * c
    cb = c.astype(jnp.bfloat16)
    fb = f.astype(jnp.bfloat16)
    one = jnp.bfloat16(1.0)
    zero = jnp.bfloat16(0.0)
    u_rows = []
    v_rows = []
    for g in range(8):
        cg = cb[g:g + 1, :]
        fg = fb[g:g + 1, :]
        u_rows.append(jnp.where(cg == iota_s, one, zero))
        v_rows.append(jnp.where(fg == iota_s, one, zero))
    u = jnp.concatenate(u_rows, axis=0).astype(jnp.float8_e4m3fn)
    v = jnp.concatenate(v_rows, axis=0).astype(jnp.float8_e4m3fn)
    return u, v


def _mxu_accumulate(u, v, mxu_index):
    w = u.shape[1]
    for k in range(w // 256):
        vk = v[:, k * 256:(k + 1) * 256]
        uk = u[:, k * 256:(k + 1) * 256]
        sreg = k % 2
        pltpu.matmul_push_rhs(vk, sreg, mxu_index, transpose=True)
        pltpu.matmul_acc_lhs(0, uk, mxu_index, load_staged_rhs=sreg)


def _diag_hist(acc):
    h = acc[0:_NSPLIT, 0:_NSPLIT]
    for g in range(1, 8):
        lo = g * _NSPLIT
        h = h + acc[lo:lo + _NSPLIT, lo:lo + _NSPLIT]
    return h


def _shift_sum(x, axis):
    n = x.shape[axis]
    k = 1
    while k < n:
        if axis == 0:
            z = jnp.zeros((k,) + x.shape[1:], x.dtype)
            x = x + jnp.concatenate([z, x[:-k]], axis=0)
        else:
            z = jnp.zeros(x.shape[:1] + (k,), x.dtype)
            x = x + jnp.concatenate([z, x[:, :-k]], axis=1)
        k *= 2
    return x


def _cum2d(h):
    a = _shift_sum(h, 1)
    tot = a[:, _NSPLIT - 1:_NSPLIT]
    ctot = _shift_sum(tot, 0)
    return a + (ctot - tot)


def _pass_b_kernel(dp_ref, dt_ref, mn_ref, mx_ref, out_ref):
    ci = pl.program_id(1)
    nchunk = pl.num_programs(1)
    rows = dp_ref.shape[1]
    w = dp_ref.shape[2]
    n_elems = float(rows * nchunk * w)

    bi = pl.program_id(0)

    @pl.when(jnp.logical_and(bi == 0, ci == 0))
    def _():
        z = pltpu.matmul_pop(0, (256, 256), jnp.float32, 0)
        z2 = pltpu.matmul_pop(0, (256, 256), jnp.float32, 1)
        out_ref[...] = jnp.full((1, 1, 128), 0.0 * (z[0, 0] + z2[0, 0]), jnp.float32)

    mn = mn_ref[0, 0, 0]
    mx = mx_ref[0, 0, 0]
    scale = (_NB - 1.0) / jnp.maximum(mx - mn, 1e-30)
    iota_s = jax.lax.broadcasted_iota(jnp.int32, (_NSPLIT, w), 0).astype(jnp.bfloat16)

    def body(s, _):
        for j in range(2):
            r0 = (2 * s + j) * 8
            up, vp = _onehot_slab(dp_ref[0, pl.ds(r0, 8), :], mn, scale, iota_s)
            ut, vt = _onehot_slab(dt_ref[0, pl.ds(r0, 8), :], mn, scale, iota_s)
            _mxu_accumulate(up, vp, 0)
            _mxu_accumulate(ut, vt, 1)
        return 0

    jax.lax.fori_loop(0, rows // 16, body, 0)

    @pl.when(ci == nchunk - 1)
    def _():
        hp = _diag_hist(pltpu.matmul_pop(0, (256, 256), jnp.float32, 0))
        ht = _diag_hist(pltpu.matmul_pop(0, (256, 256), jnp.float32, 1))
        cump = _cum2d(hp)
        cumt = _cum2d(ht)
        fi = jax.lax.broadcasted_iota(jnp.int32, (_NSPLIT, _NSPLIT), 0).astype(jnp.float32)
        fj = jax.lax.broadcasted_iota(jnp.int32, (_NSPLIT, _NSPLIT), 1).astype(jnp.float32)
        dd = (cump - cumt) * (1.0 / n_elems)
        d2 = dd * dd
        lin = _NSPLIT * fi + fj
        total = jnp.sum(jnp.where(lin <= _NB - 1.0, d2, 0.0))
        first = jnp.sum(jnp.where(lin == 0.0, d2, 0.0))
        last = jnp.sum(jnp.where(lin == _NB - 1.0, d2, 0.0))
        dx = (mx - mn) * (1.0 / (_NB - 1.0))
        crps = dx * (total - 0.5 * (first + last))
        out_ref[...] = jnp.full((1, 1, 128), crps, jnp.float32)


def kernel(prediction, target):
    b, _, h, w = prediction.shape
    p = prediction.reshape(b, h, w)
    t = target.reshape(b, h, w)

    dp, dt, mn, mx = pl.pallas_call(
        _pass_a_kernel,
        out_shape=(
            jax.ShapeDtypeStruct((b, h, w), jnp.float32),
            jax.ShapeDtypeStruct((b, h, w), jnp.float32),
            jax.ShapeDtypeStruct((b, 1, 128), jnp.float32),
            jax.ShapeDtypeStruct((b, 1, 128), jnp.float32),
        ),
        grid=(b,),
        in_specs=[
            pl.BlockSpec((1, h, w), lambda i: (i, 0, 0)),
            pl.BlockSpec((1, h, w), lambda i: (i, 0, 0)),
        ],
        out_specs=(
            pl.BlockSpec((1, h, w), lambda i: (i, 0, 0)),
            pl.BlockSpec((1, h, w), lambda i: (i, 0, 0)),
            pl.BlockSpec((1, 1, 128), lambda i: (i, 0, 0)),
            pl.BlockSpec((1, 1, 128), lambda i: (i, 0, 0)),
        ),
        compiler_params=pltpu.CompilerParams(
            dimension_semantics=("parallel",),
            vmem_limit_bytes=56 * 1024 * 1024,
        ),
        name="crps_pool_dist",
    )(p, t)

    rows = 128
    nchunk = h // rows
    crps = pl.pallas_call(
        _pass_b_kernel,
        out_shape=jax.ShapeDtypeStruct((b, 1, 128), jnp.float32),
        grid=(b, nchunk),
        in_specs=[
            pl.BlockSpec((1, rows, w), lambda i, j: (i, j, 0)),
            pl.BlockSpec((1, rows, w), lambda i, j: (i, j, 0)),
            pl.BlockSpec((1, 1, 128), lambda i, j: (i, 0, 0)),
            pl.BlockSpec((1, 1, 128), lambda i, j: (i, 0, 0)),
        ],
        out_specs=pl.BlockSpec((1, 1, 128), lambda i, j: (i, 0, 0)),
        compiler_params=pltpu.CompilerParams(
            dimension_semantics=("parallel", "arbitrary"),
            vmem_limit_bytes=56 * 1024 * 1024,
        ),
        name="crps_hist",
    )(dp, dt, mn, mx)

    return jnp.mean(crps[:, 0, 0])


# speedup vs baseline: 1.3377x; 1.0084x over previous
import jax
import jax.numpy as jnp
from jax.experimental import pallas as pl
from jax.experimental.pallas import tpu as pltpu

_NB = 1000
_NSPLIT = 32
_PAD = 4


def _box9(x, axis):
    n = x.shape[axis]
    zshape = list(x.shape)
    zshape[axis] = _PAD
    z = jnp.zeros(zshape, x.dtype)
    xp = jnp.concatenate([z, x, z], axis=axis)

    def sl(a, lo, hi):
        if axis == 0:
            return a[lo:hi]
        return a[:, lo:hi]

    m = n + 8
    w2 = sl(xp, 0, m - 1) + sl(xp, 1, m)
    w4 = sl(w2, 0, m - 3) + sl(w2, 2, m - 1)
    w8 = sl(w4, 0, m - 7) + sl(w4, 4, m - 3)
    return sl(w8, 0, n) + sl(xp, 8, m)


def _dist(x):
    h, w = x.shape
    s = _box9(_box9(x, 0), 1)
    ri = jax.lax.broadcasted_iota(jnp.int32, (h, w), 0).astype(jnp.float32)
    ci = jax.lax.broadcasted_iota(jnp.int32, (h, w), 1).astype(jnp.float32)
    cr = jnp.minimum(ri + 4.0, h - 1.0) - jnp.maximum(ri - 4.0, 0.0) + 1.0
    cc = jnp.minimum(ci + 4.0, w - 1.0) - jnp.maximum(ci - 4.0, 0.0) + 1.0
    return x - s / (cr * cc)


def _pass_a_kernel(p_ref, t_ref, dp_ref, dt_ref, mn_ref, mx_ref):
    dp = _dist(p_ref[0])
    dt = _dist(t_ref[0])
    dp_ref[0] = dp
    dt_ref[0] = dt
    mn = jnp.minimum(jnp.min(dp), jnp.min(dt))
    mx = jnp.maximum(jnp.max(dp), jnp.max(dt))
    mn_ref[...] = jnp.full((1, 1, 128), mn, jnp.float32)
    mx_ref[...] = jnp.full((1, 1, 128), mx, jnp.float32)


def _onehot_slab(x, mn, scale, iota_s):
    t = (x - mn) * scale
    bidx = jnp.clip(jnp.ceil(t), 0.0, _NB - 1.0)
    c = jnp.floor(bidx * (1.0 / _NSPLIT))
    f = bidx - _NSPLIT * c
    cb = c.astype(jnp.bfloat16)
    fb = f.astype(jnp.bfloat16)
    one = jnp.bfloat16(1.0)
    zero = jnp.bfloat16(0.0)
    u_rows = []
    v_rows = []
    for g in range(8):
        cg = cb[g:g + 1, :]
        fg = fb[g:g + 1, :]
        u_rows.append(jnp.where(cg == iota_s, one, zero))
        v_rows.append(jnp.where(fg == iota_s, one, zero))
    u = jnp.concatenate(u_rows, axis=0).astype(jnp.float8_e4m3fn)
    v = jnp.concatenate(v_rows, axis=0).astype(jnp.float8_e4m3fn)
    return u, v


def _mxu_accumulate(u, v, mxu_index):
    w = u.shape[1]
    for k in range(w // 256):
        vk = v[:, k * 256:(k + 1) * 256]
        uk = u[:, k * 256:(k + 1) * 256]
        sreg = k % 2
        pltpu.matmul_push_rhs(vk, sreg, mxu_index, transpose=True)
        pltpu.matmul_acc_lhs(0, uk, mxu_index, load_staged_rhs=sreg)


def _diag_hist(acc):
    h = acc[0:_NSPLIT, 0:_NSPLIT]
    for g in range(1, 8):
        lo = g * _NSPLIT
        h = h + acc[lo:lo + _NSPLIT, lo:lo + _NSPLIT]
    return h


def _shift_sum(x, axis):
    n = x.shape[axis]
    k = 1
    while k < n:
        if axis == 0:
            z = jnp.zeros((k,) + x.shape[1:], x.dtype)
            x = x + jnp.concatenate([z, x[:-k]], axis=0)
        else:
            z = jnp.zeros(x.shape[:1] + (k,), x.dtype)
            x = x + jnp.concatenate([z, x[:, :-k]], axis=1)
        k *= 2
    return x


def _cum2d(h):
    a = _shift_sum(h, 1)
    tot = a[:, _NSPLIT - 1:_NSPLIT]
    ctot = _shift_sum(tot, 0)
    return a + (ctot - tot)


def _pass_b_kernel(dp_ref, dt_ref, mn_ref, mx_ref, out_ref):
    ci = pl.program_id(1)
    nchunk = pl.num_programs(1)
    rows = dp_ref.shape[1]
    w = dp_ref.shape[2]
    n_elems = float(rows * nchunk * w)

    bi = pl.program_id(0)

    @pl.when(jnp.logical_and(bi == 0, ci == 0))
    def _():
        z = pltpu.matmul_pop(0, (256, 256), jnp.float32, 0)
        z2 = pltpu.matmul_pop(0, (256, 256), jnp.float32, 1)
        out_ref[...] = jnp.full((1, 1, 128), 0.0 * (z[0, 0] + z2[0, 0]), jnp.float32)

    mn = mn_ref[0, 0, 0]
    mx = mx_ref[0, 0, 0]
    scale = (_NB - 1.0) / jnp.maximum(mx - mn, 1e-30)
    iota_s = jax.lax.broadcasted_iota(jnp.int32, (_NSPLIT, w), 0).astype(jnp.bfloat16)

    def body(s, _):
        for j in range(4):
            r0 = (4 * s + j) * 8
            up, vp = _onehot_slab(dp_ref[0, pl.ds(r0, 8), :], mn, scale, iota_s)
            ut, vt = _onehot_slab(dt_ref[0, pl.ds(r0, 8), :], mn, scale, iota_s)
            _mxu_accumulate(up, vp, 0)
            _mxu_accumulate(ut, vt, 1)
        return 0

    jax.lax.fori_loop(0, rows // 32, body, 0)

    @pl.when(ci == nchunk - 1)
    def _():
        hp = _diag_hist(pltpu.matmul_pop(0, (256, 256), jnp.float32, 0))
        ht = _diag_hist(pltpu.matmul_pop(0, (256, 256), jnp.float32, 1))
        cump = _cum2d(hp)
        cumt = _cum2d(ht)
        fi = jax.lax.broadcasted_iota(jnp.int32, (_NSPLIT, _NSPLIT), 0).astype(jnp.float32)
        fj = jax.lax.broadcasted_iota(jnp.int32, (_NSPLIT, _NSPLIT), 1).astype(jnp.float32)
        dd = (cump - cumt) * (1.0 / n_elems)
        d2 = dd * dd
        lin = _NSPLIT * fi + fj
        total = jnp.sum(jnp.where(lin <= _NB - 1.0, d2, 0.0))
        first = jnp.sum(jnp.where(lin == 0.0, d2, 0.0))
        last = jnp.sum(jnp.where(lin == _NB - 1.0, d2, 0.0))
        dx = (mx - mn) * (1.0 / (_NB - 1.0))
        crps = dx * (total - 0.5 * (first + last))
        out_ref[...] = jnp.full((1, 1, 128), crps, jnp.float32)


def kernel(prediction, target):
    b, _, h, w = prediction.shape
    p = prediction.reshape(b, h, w)
    t = target.reshape(b, h, w)

    dp, dt, mn, mx = pl.pallas_call(
        _pass_a_kernel,
        out_shape=(
            jax.ShapeDtypeStruct((b, h, w), jnp.float32),
            jax.ShapeDtypeStruct((b, h, w), jnp.float32),
            jax.ShapeDtypeStruct((b, 1, 128), jnp.float32),
            jax.ShapeDtypeStruct((b, 1, 128), jnp.float32),
        ),
        grid=(b,),
        in_specs=[
            pl.BlockSpec((1, h, w), lambda i: (i, 0, 0)),
            pl.BlockSpec((1, h, w), lambda i: (i, 0, 0)),
        ],
        out_specs=(
            pl.BlockSpec((1, h, w), lambda i: (i, 0, 0)),
            pl.BlockSpec((1, h, w), lambda i: (i, 0, 0)),
            pl.BlockSpec((1, 1, 128), lambda i: (i, 0, 0)),
            pl.BlockSpec((1, 1, 128), lambda i: (i, 0, 0)),
        ),
        compiler_params=pltpu.CompilerParams(
            dimension_semantics=("parallel",),
            vmem_limit_bytes=56 * 1024 * 1024,
        ),
        name="crps_pool_dist",
    )(p, t)

    rows = 128
    nchunk = h // rows
    crps = pl.pallas_call(
        _pass_b_kernel,
        out_shape=jax.ShapeDtypeStruct((b, 1, 128), jnp.float32),
        grid=(b, nchunk),
        in_specs=[
            pl.BlockSpec((1, rows, w), lambda i, j: (i, j, 0)),
            pl.BlockSpec((1, rows, w), lambda i, j: (i, j, 0)),
            pl.BlockSpec((1, 1, 128), lambda i, j: (i, 0, 0)),
            pl.BlockSpec((1, 1, 128), lambda i, j: (i, 0, 0)),
        ],
        out_specs=pl.BlockSpec((1, 1, 128), lambda i, j: (i, 0, 0)),
        compiler_params=pltpu.CompilerParams(
            dimension_semantics=("parallel", "arbitrary"),
            vmem_limit_bytes=56 * 1024 * 1024,
        ),
        name="crps_hist",
    )(dp, dt, mn, mx)

    return jnp.mean(crps[:, 0, 0])
